# baseline (speedup 1.0000x reference)
import functools

import jax
import jax.numpy as jnp
from jax import lax
from jax.experimental import pallas as pl
from jax.experimental.pallas import tpu as pltpu
from jax.experimental.pallas import tpu_sc as plsc

_N = 100000
_D = 256
_G = 128
_NCORES = 2
_NSUB = 16
_NW = _NCORES * _NSUB
_RPT = _N // _NW
_IPAD = 3136
_CHUNK = 64
_CSZ = _CHUNK * _D
_NBUF = 3
_NFULL = _RPT // _CHUNK
_TAIL = _RPT - _NFULL * _CHUNK
_TFULL = _TAIL // 16
_TREM = _TAIL - _TFULL * 16
_NVR = _D // 16
_AW = _D + 16
_ASZ = _G * _AW


def _sc_segment_sum(nodef, ids2d):
    mesh = plsc.VectorSubcoreMesh(core_axis_name="c", subcore_axis_name="s")

    @functools.partial(
        pl.kernel,
        mesh=mesh,
        out_type=jax.ShapeDtypeStruct((_NCORES, _NSUB, _ASZ), jnp.float32),
        scratch_types=[
            pltpu.VMEM((_IPAD,), jnp.int32),
            pltpu.VMEM((_CSZ,), jnp.float32),
            pltpu.VMEM((_CSZ,), jnp.float32),
            pltpu.VMEM((_CSZ,), jnp.float32),
            pltpu.VMEM((_AW,), jnp.float32),
            pltpu.VMEM((_ASZ,), jnp.float32),
            pltpu.SemaphoreType.DMA,
            pltpu.SemaphoreType.DMA,
            pltpu.SemaphoreType.DMA,
        ],
    )
    def k(node_hbm, ids_hbm, out_acc,
          idx_v, buf_0, buf_1, buf_2, rbuf_v, acc_v, sem_0, sem_1, sem_2):
        c = lax.axis_index("c")
        s = lax.axis_index("s")
        wid = c * _NSUB + s
        woff = pl.multiple_of(wid * (_RPT * _D), _D)
        bufs = [buf_0, buf_1, buf_2]
        sems = [sem_0, sem_1, sem_2]

        def _zrow(i, carry):
            for j in range(_NVR + 1):
                acc_v[pl.ds(i * _AW + j * 16, 16)] = \
                    jnp.zeros((16,), jnp.float32)
            return carry
        lax.fori_loop(0, _G, _zrow, 0)

        pltpu.sync_copy(ids_hbm.at[wid], idx_v)
        for b in range(_NBUF):
            pltpu.async_copy(node_hbm.at[pl.ds(woff + b * _CSZ, _CSZ)],
                             bufs[b], sems[b])

        ones = jnp.ones((16,), jnp.float32)
        chunk_cnt = jnp.full((16,), float(_CHUNK), jnp.float32)
        zero16 = jnp.zeros((16,), jnp.float32)

        def _flush_to(prev, regs):
            b = prev * _AW
            for j in range(_NVR + 1):
                acc_v[pl.ds(b + j * 16, 16)] = regs[j]

        def _rows(buf, boff, sidv, lanes, carry):
            prev = carry[0]
            regs = list(carry[1:])
            for kk, lane in lanes:
                sid = sidv[lane]
                base = boff + kk * _D

                def _flush(prev=prev, regs=regs):
                    _flush_to(prev, regs)
                    return (zero16,) * (_NVR + 1)

                def _keep(regs=regs):
                    return tuple(regs)

                regs = list(lax.cond(sid != prev, _flush, _keep))
                for j in range(_NVR):
                    regs[j] = regs[j] + buf[pl.ds(base + j * 16, 16)]
                regs[_NVR] = regs[_NVR] + ones
                prev = sid
            return (prev, *regs)

        def _rload():
            return [rbuf_v[pl.ds(j * 16, 16)] for j in range(_NVR + 1)]

        def _rstore(regs):
            for j in range(_NVR + 1):
                rbuf_v[pl.ds(j * 16, 16)] = regs[j]

        def _chunk(t, buf, prev):
            ioff = t * _CHUNK
            cfirst = idx_v[pl.ds(ioff, 16)][0]
            clast = idx_v[pl.ds(ioff + _CHUNK - 16, 16)][15]

            def _fast(prev):
                regs = _rload()

                def _flush(regs=regs):
                    _flush_to(prev, regs)
                    return (zero16,) * (_NVR + 1)

                def _keep(regs=regs):
                    return tuple(regs)

                regs = lax.cond(cfirst != prev, _flush, _keep)

                def _fg(g, regs_t):
                    regs_l = list(regs_t)
                    base0 = g * 16 * _D
                    for kk in range(16):
                        base = base0 + kk * _D
                        for j in range(_NVR):
                            regs_l[j] = regs_l[j] + buf[pl.ds(base + j * 16,
                                                              16)]
                    return tuple(regs_l)

                regs = list(lax.fori_loop(0, _CHUNK // 16, _fg, regs))
                regs[_NVR] = regs[_NVR] + chunk_cnt
                _rstore(regs)
                return clast

            def _slow(prev):
                carry = (prev, *_rload())

                def _sg(g, carry):
                    sidv = idx_v[pl.ds(ioff + g * 16, 16)]
                    return _rows(buf, g * 16 * _D, sidv,
                                 [(kk, kk) for kk in range(16)], carry)

                carry = lax.fori_loop(0, _CHUNK // 16, _sg, carry)
                _rstore(list(carry[1:]))
                return carry[0]

            return lax.cond(cfirst == clast, _fast, _slow, prev)

        sid0 = idx_v[pl.ds(0, 16)][0]
        _rstore([zero16] * (_NVR + 1))
        prev = sid0

        def _ring(m, prev):
            for b in range(_NBUF):
                t = _NBUF * m + b
                pltpu.make_async_copy(
                    node_hbm.at[pl.ds(woff + t * _CSZ, _CSZ)],
                    bufs[b], sems[b]).wait()
                prev = _chunk(t, bufs[b], prev)

                @pl.when(m < _NFULL // _NBUF - 1)
                def _():
                    pltpu.async_copy(
                        node_hbm.at[pl.ds(woff + (t + _NBUF) * _CSZ, _CSZ)],
                        bufs[b], sems[b])
            return prev

        prev = lax.fori_loop(0, _NFULL // _NBUF, _ring, prev)

        pltpu.sync_copy(node_hbm.at[pl.ds(woff + _NFULL * _CSZ, _TAIL * _D)],
                        buf_0.at[pl.ds(0, _TAIL * _D)])
        carry = (prev, *_rload())

        def _tg(g, carry):
            sidv = idx_v[pl.ds(_NFULL * _CHUNK + g * 16, 16)]
            return _rows(buf_0, g * 16 * _D, sidv,
                         [(kk, kk) for kk in range(16)], carry)
        carry = lax.fori_loop(0, _TFULL, _tg, carry)

        sidv_t = idx_v[pl.ds(_IPAD - 16, 16)]
        carry = _rows(buf_0, _TFULL * 16 * _D, sidv_t,
                      [(kk, kk) for kk in range(_TREM)], carry)

        _flush_to(carry[0], carry[1:])
        pltpu.sync_copy(acc_v, out_acc.at[c, s])

    return k(nodef, ids2d)


def _dense_body(pacc_ref, w1_ref, b1_ref, w2_ref, b2_ref, logp_ref, gs_ref):
    pacc = pacc_ref[...]
    part = jnp.sum(pacc, axis=0)
    seg = part[:, :_D]
    cnt = part[:, _D:_D + 1]
    gs = lax.dot(seg, w1_ref[...], precision=lax.Precision.HIGHEST)
    gs = gs + cnt * b1_ref[...]
    logits = lax.dot(gs, w2_ref[...], precision=lax.Precision.HIGHEST)
    logits = logits + b2_ref[...]
    m = jnp.max(logits, axis=1, keepdims=True)
    lse = m + jnp.log(jnp.sum(jnp.exp(logits - m), axis=1, keepdims=True))
    logp_ref[...] = logits - lse
    gs_ref[...] = gs


def kernel(node_features, batch_segments, num_graphs, W1, b1, W2, b2):
    del num_graphs
    d_hid = W1.shape[1]
    n_cls = W2.shape[1]
    nodef = node_features.reshape(_N * _D)
    ids2d = batch_segments.astype(jnp.int32).reshape(_NW, _RPT)
    ids2d = jnp.pad(ids2d, ((0, 0), (0, _IPAD - _RPT)))

    pacc = _sc_segment_sum(nodef, ids2d)
    pacc = pacc.reshape(_NW, _G, _AW)

    logp, gs = pl.pallas_call(
        _dense_body,
        out_shape=(
            jax.ShapeDtypeStruct((_G, n_cls), jnp.float32),
            jax.ShapeDtypeStruct((_G, d_hid), jnp.float32),
        ),
    )(pacc, W1, b1.reshape(1, d_hid), W2, b2.reshape(1, n_cls))
    return (logp, gs)


# speedup vs baseline: 1.9582x; 1.9582x over previous
import functools

import jax
import jax.numpy as jnp
from jax import lax
from jax.experimental import pallas as pl
from jax.experimental.pallas import tpu as pltpu
from jax.experimental.pallas import tpu_sc as plsc

_N = 100000
_D = 256
_G = 128
_NCORES = 2
_NSUB = 16
_NW = _NCORES * _NSUB
_RPT = _N // _NW
_IPAD = 3136
_CHUNK = 64
_CSZ = _CHUNK * _D
_NBUF = 3
_NFULL = _RPT // _CHUNK
_TAIL = _RPT - _NFULL * _CHUNK
_TFULL = _TAIL // 16
_TREM = _TAIL - _TFULL * 16
_NVR = _D // 16
_AW = _D + 16
_ASZ = _G * _AW


def _sc_segment_sum(nodef, ids2d):
    mesh = plsc.VectorSubcoreMesh(core_axis_name="c", subcore_axis_name="s")

    @functools.partial(
        pl.kernel,
        mesh=mesh,
        out_type=jax.ShapeDtypeStruct((_NCORES, _NSUB, _ASZ), jnp.float32),
        scratch_types=[
            pltpu.VMEM((_IPAD,), jnp.int32),
            pltpu.VMEM((_CSZ,), jnp.float32),
            pltpu.VMEM((_CSZ,), jnp.float32),
            pltpu.VMEM((_CSZ,), jnp.float32),
            pltpu.VMEM((_AW,), jnp.float32),
            pltpu.VMEM((_ASZ,), jnp.float32),
            pltpu.SemaphoreType.DMA,
            pltpu.SemaphoreType.DMA,
            pltpu.SemaphoreType.DMA,
        ],
    )
    def k(node_hbm, ids_hbm, out_acc,
          idx_v, buf_0, buf_1, buf_2, rbuf_v, acc_v, sem_0, sem_1, sem_2):
        c = lax.axis_index("c")
        s = lax.axis_index("s")
        wid = c * _NSUB + s
        woff = pl.multiple_of(wid * (_RPT * _D), _D)
        bufs = [buf_0, buf_1, buf_2]
        sems = [sem_0, sem_1, sem_2]

        def _zrow(i, carry):
            for j in range(_NVR + 1):
                acc_v[pl.ds(i * _AW + j * 16, 16)] = \
                    jnp.zeros((16,), jnp.float32)
            return carry
        lax.fori_loop(0, _G, _zrow, 0)

        pltpu.sync_copy(ids_hbm.at[wid], idx_v)
        for b in range(_NBUF):
            pltpu.async_copy(node_hbm.at[pl.ds(woff + b * _CSZ, _CSZ)],
                             bufs[b], sems[b])

        ones = jnp.ones((16,), jnp.float32)
        chunk_cnt = jnp.full((16,), float(_CHUNK), jnp.float32)
        zero16 = jnp.zeros((16,), jnp.float32)

        def _flush_to(prev, regs):
            b = prev * _AW
            for j in range(_NVR + 1):
                acc_v[pl.ds(b + j * 16, 16)] = regs[j]

        def _rows(buf, boff, sidv, lanes, carry):
            prev = carry[0]
            regs = list(carry[1:])
            for kk, lane in lanes:
                sid = sidv[lane]
                base = boff + kk * _D

                def _flush(prev=prev, regs=regs):
                    _flush_to(prev, regs)
                    return (zero16,) * (_NVR + 1)

                def _keep(regs=regs):
                    return tuple(regs)

                regs = list(lax.cond(sid != prev, _flush, _keep))
                for j in range(_NVR):
                    regs[j] = regs[j] + buf[pl.ds(base + j * 16, 16)]
                regs[_NVR] = regs[_NVR] + ones
                prev = sid
            return (prev, *regs)

        def _rload():
            return [rbuf_v[pl.ds(j * 16, 16)] for j in range(_NVR + 1)]

        def _rstore(regs):
            for j in range(_NVR + 1):
                rbuf_v[pl.ds(j * 16, 16)] = regs[j]

        def _chunk(t, buf, prev):
            ioff = t * _CHUNK
            cfirst = idx_v[pl.ds(ioff, 16)][0]
            clast = idx_v[pl.ds(ioff + _CHUNK - 16, 16)][15]

            def _fast(prev):
                regs = _rload()

                def _flush(regs=regs):
                    _flush_to(prev, regs)
                    return (zero16,) * (_NVR + 1)

                def _keep(regs=regs):
                    return tuple(regs)

                regs = lax.cond(cfirst != prev, _flush, _keep)

                def _fg(g, regs_t):
                    regs_l = list(regs_t)
                    base0 = g * 4 * _D
                    for kk in range(4):
                        base = base0 + kk * _D
                        for j in range(_NVR):
                            regs_l[j] = regs_l[j] + buf[pl.ds(base + j * 16,
                                                              16)]
                    return tuple(regs_l)

                regs = list(lax.fori_loop(0, _CHUNK // 4, _fg, regs))
                regs[_NVR] = regs[_NVR] + chunk_cnt
                _rstore(regs)
                return clast

            def _slow(prev):
                carry = (prev, *_rload())

                def _sg(g, carry):
                    sidv = idx_v[pl.ds(ioff + g * 16, 16)]
                    return _rows(buf, g * 16 * _D, sidv,
                                 [(kk, kk) for kk in range(16)], carry)

                carry = lax.fori_loop(0, _CHUNK // 16, _sg, carry)
                _rstore(list(carry[1:]))
                return carry[0]

            return lax.cond(cfirst == clast, _fast, _slow, prev)

        sid0 = idx_v[pl.ds(0, 16)][0]
        _rstore([zero16] * (_NVR + 1))
        prev = sid0

        def _ring(m, prev):
            for b in range(_NBUF):
                t = _NBUF * m + b
                pltpu.make_async_copy(
                    node_hbm.at[pl.ds(woff + t * _CSZ, _CSZ)],
                    bufs[b], sems[b]).wait()
                prev = _chunk(t, bufs[b], prev)

                @pl.when(m < _NFULL // _NBUF - 1)
                def _():
                    pltpu.async_copy(
                        node_hbm.at[pl.ds(woff + (t + _NBUF) * _CSZ, _CSZ)],
                        bufs[b], sems[b])
            return prev

        prev = lax.fori_loop(0, _NFULL // _NBUF, _ring, prev)

        pltpu.sync_copy(node_hbm.at[pl.ds(woff + _NFULL * _CSZ, _TAIL * _D)],
                        buf_0.at[pl.ds(0, _TAIL * _D)])
        carry = (prev, *_rload())

        def _tg(g, carry):
            sidv = idx_v[pl.ds(_NFULL * _CHUNK + g * 16, 16)]
            return _rows(buf_0, g * 16 * _D, sidv,
                         [(kk, kk) for kk in range(16)], carry)
        carry = lax.fori_loop(0, _TFULL, _tg, carry)

        sidv_t = idx_v[pl.ds(_IPAD - 16, 16)]
        carry = _rows(buf_0, _TFULL * 16 * _D, sidv_t,
                      [(kk, kk) for kk in range(_TREM)], carry)

        _flush_to(carry[0], carry[1:])
        pltpu.sync_copy(acc_v, out_acc.at[c, s])

    return k(nodef, ids2d)


def _dense_body(pacc_ref, w1_ref, b1_ref, w2_ref, b2_ref, logp_ref, gs_ref):
    pacc = pacc_ref[...]
    part = jnp.sum(pacc, axis=0)
    seg = part[:, :_D]
    cnt = part[:, _D:_D + 1]
    gs = lax.dot(seg, w1_ref[...], precision=lax.Precision.HIGHEST)
    gs = gs + cnt * b1_ref[...]
    logits = lax.dot(gs, w2_ref[...], precision=lax.Precision.HIGHEST)
    logits = logits + b2_ref[...]
    m = jnp.max(logits, axis=1, keepdims=True)
    lse = m + jnp.log(jnp.sum(jnp.exp(logits - m), axis=1, keepdims=True))
    logp_ref[...] = logits - lse
    gs_ref[...] = gs


def kernel(node_features, batch_segments, num_graphs, W1, b1, W2, b2):
    del num_graphs
    d_hid = W1.shape[1]
    n_cls = W2.shape[1]
    nodef = node_features.reshape(_N * _D)
    ids2d = batch_segments.astype(jnp.int32).reshape(_NW, _RPT)
    ids2d = jnp.pad(ids2d, ((0, 0), (0, _IPAD - _RPT)))

    pacc = _sc_segment_sum(nodef, ids2d)
    pacc = pacc.reshape(_NW, _G, _AW)

    logp, gs = pl.pallas_call(
        _dense_body,
        out_shape=(
            jax.ShapeDtypeStruct((_G, n_cls), jnp.float32),
            jax.ShapeDtypeStruct((_G, d_hid), jnp.float32),
        ),
    )(pacc, W1, b1.reshape(1, d_hid), W2, b2.reshape(1, n_cls))
    return (logp, gs)
